# baseline (speedup 1.0000x reference)
import functools

import jax
import jax.numpy as jnp
from jax import lax
from jax.experimental import pallas as pl
from jax.experimental.pallas import tpu as pltpu

D_MODEL = 1024
MAXLEN = 8192
BASE = 10000.0
TOTAL = 4 * 8192
R = 256
GRID = TOTAL // R
UNROLL = 8


def _tc_body(ids_ref, freq_ref, out_ref):
    def body(j, carry):
        base = j * UNROLL
        for u in range(UNROLL):
            r = base + u
            p = ids_ref[0, 0, r].astype(jnp.float32)
            out_ref[r] = jnp.sin(p * freq_ref[0] + freq_ref[1])
        return carry

    lax.fori_loop(0, R // UNROLL, body, 0)


@jax.jit
def tc_sin(position_ids, pe):
    idx = position_ids.reshape(GRID, 1, R).astype(jnp.int32)
    dim_indices = jnp.arange(0, D_MODEL, 2, dtype=jnp.float32)
    frequencies = 1.0 / (BASE ** (dim_indices / D_MODEL))
    freq_full = jnp.repeat(frequencies, 2).reshape(1, 8, 128)
    phase_full = jnp.tile(
        jnp.array([0.0, jnp.pi / 2], dtype=jnp.float32), D_MODEL // 2
    ).reshape(1, 8, 128)
    fp = jnp.concatenate([freq_full, phase_full], axis=0)
    out = pl.pallas_call(
        _tc_body,
        grid=(GRID,),
        in_specs=[
            pl.BlockSpec((1, 1, R), lambda i: (i, 0, 0), memory_space=pltpu.SMEM),
            pl.BlockSpec((2, 8, 128), lambda i: (0, 0, 0)),
        ],
        out_specs=pl.BlockSpec((R, 8, 128), lambda i: (i, 0, 0)),
        out_shape=jax.ShapeDtypeStruct((TOTAL, 8, 128), jnp.float32),
    )(idx, fp)
    return out.reshape(position_ids.shape + (D_MODEL,))


def kernel(position_ids, pe):
    return tc_sin(position_ids, pe)


# speedup vs baseline: 5.8108x; 5.8108x over previous
import functools

import jax
import jax.numpy as jnp
from jax import lax
from jax.experimental import pallas as pl
from jax.experimental.pallas import tpu as pltpu
from jax.experimental.pallas import tpu_sc as plsc

D_MODEL = 1024
NUM_CORES = 2
NUM_SUBCORES = 16
NUM_WORKERS = NUM_CORES * NUM_SUBCORES
TOTAL = 4 * 8192
PER_WORKER = TOTAL // NUM_WORKERS
CHUNK = 16
NUM_BUFS = 4
NUM_CHUNKS = PER_WORKER // CHUNK
NUM_ROUNDS = NUM_CHUNKS // NUM_BUFS

_mesh = plsc.VectorSubcoreMesh(core_axis_name="c", subcore_axis_name="s")


@functools.partial(
    pl.kernel,
    mesh=_mesh,
    out_type=jax.ShapeDtypeStruct((TOTAL, D_MODEL), jnp.float32),
    scratch_types=[
        pltpu.VMEM((NUM_CHUNKS, CHUNK), jnp.int32),
        pltpu.VMEM((NUM_BUFS, CHUNK, D_MODEL), jnp.float32),
        pltpu.SemaphoreType.DMA,
        pltpu.SemaphoreType.DMA,
        pltpu.SemaphoreType.DMA,
        pltpu.SemaphoreType.DMA,
        pltpu.SemaphoreType.DMA,
        pltpu.SemaphoreType.DMA,
        pltpu.SemaphoreType.DMA,
        pltpu.SemaphoreType.DMA,
    ],
)
def _gather_kernel(pe_hbm, idx_hbm, out_hbm, idx_v, bufs, *sems):
    gsem = sems[:NUM_BUFS]
    ssem = sems[NUM_BUFS:]
    wid = lax.axis_index("s") * NUM_CORES + lax.axis_index("c")
    base = wid * PER_WORKER
    pltpu.sync_copy(idx_hbm.at[wid], idx_v)

    def start_gather(c, b):
        pltpu.async_copy(pe_hbm.at[idx_v.at[c]], bufs.at[b], gsem[b])

    def wait_gather(c, b):
        pltpu.make_async_copy(pe_hbm.at[idx_v.at[c]], bufs.at[b], gsem[b]).wait()

    def start_scatter(c, b):
        pltpu.async_copy(
            bufs.at[b], out_hbm.at[pl.ds(base + c * CHUNK, CHUNK)], ssem[b]
        )

    def wait_scatter(c, b):
        pltpu.make_async_copy(
            bufs.at[b], out_hbm.at[pl.ds(base + c * CHUNK, CHUNK)], ssem[b]
        ).wait()

    for b in range(NUM_BUFS):
        start_gather(b, b)

    def body(i, carry):
        c0 = i * NUM_BUFS
        for b in range(NUM_BUFS):
            wait_gather(c0 + b, b)
            start_scatter(c0 + b, b)
        for b in range(NUM_BUFS):
            wait_scatter(c0 + b, b)
            start_gather(c0 + NUM_BUFS + b, b)
        return carry

    lax.fori_loop(0, NUM_ROUNDS - 1, body, 0)

    last = (NUM_ROUNDS - 1) * NUM_BUFS
    for b in range(NUM_BUFS):
        wait_gather(last + b, b)
        start_scatter(last + b, b)
    for b in range(NUM_BUFS):
        wait_scatter(last + b, b)


def kernel(position_ids, pe):
    idx = position_ids.reshape(NUM_WORKERS, NUM_CHUNKS, CHUNK).astype(jnp.int32)
    out = _gather_kernel(pe, idx)
    return out.reshape(position_ids.shape + (D_MODEL,))
